# Initial kernel scaffold; baseline (speedup 1.0000x reference)
import jax
import jax.numpy as jnp
from jax.experimental import pallas as pl


def kernel(x_nchw, w1k, b1k, w2k, b2k, w3k, b3k, w4k, b4k, wf1k, bf1k, wf2k, bf2k):
    raise NotImplementedError("write your pallas kernel here")


# speedup vs baseline: 31.4193x; 31.4193x over previous
import numpy as np
import jax
import jax.numpy as jnp
from jax import lax
from jax.experimental import pallas as pl
from jax.experimental.pallas import tpu as pltpu

F32 = jnp.float32


def _band(wk, cin, cout, w_in_pad, w_out_pad, pw, k_piece_pad=None):
    k4 = wk.reshape(3, 3, cin, cout)
    sel = np.zeros((3, w_in_pad, w_out_pad), np.float32)
    for dw in range(3):
        for wp in range(w_out_pad):
            w = 2 * wp + pw + dw
            if w < w_in_pad:
                sel[dw, w, wp] = 1.0
    sel = jnp.asarray(sel)
    b = jnp.einsum('awp,haio->hwipo', sel, k4)
    b = b.reshape(3, w_in_pad * cin, w_out_pad * cout)
    if k_piece_pad is not None:
        b = jnp.pad(b, ((0, 0), (0, k_piece_pad - w_in_pad * cin), (0, 0)))
    return b.reshape(-1, w_out_pad * cout)


def _net_kernel(x_ref,
                b1a_ref, b1b_ref, t1_ref,
                b2a_ref, b2b_ref, t2_ref,
                b3a_ref, b3b_ref, t3_ref,
                b4a_ref, b4b_ref, t4_ref,
                wf1_ref, bf1_ref, wf2_ref, bf2_ref,
                out_ref,
                x1s, x2s, x3s, x4s, ps):
    nb8 = x1s.shape[1] - 8
    nb = nb8 // 64

    x1s[:, 0:nb8, :] = x_ref[...]
    x1s[:, nb8:nb8 + 8, :] = jnp.zeros((2, 8, 128), F32)

    def stage(src, dst, ba_ref, bb_ref, bt_ref):
        ng = src.shape[0]
        mh = (src.shape[1] - 8) // 2

        def lhs_for(ph):
            return jnp.concatenate(
                [src[c, pl.Slice(ph + dh, mh, 2), :]
                 for dh in range(3) for c in range(ng)], axis=1)

        lhs = jnp.concatenate([lhs_for(0), lhs_for(1)], axis=0)
        bt = bt_ref[...]
        acta = jnp.maximum(jnp.dot(lhs, ba_ref[...], preferred_element_type=F32) + bt, 0.0)
        actb = jnp.maximum(jnp.dot(lhs, bb_ref[...], preferred_element_type=F32) + bt, 0.0)
        act = jnp.maximum(acta, actb)
        pooled = jnp.maximum(act[0:mh, :], act[mh:2 * mh, :])
        for c in range(dst.shape[0]):
            dst[c, 0:mh, :] = pooled[:, c * 128:(c + 1) * 128]
            if dst.shape[1] > mh:
                dst[c, mh:, :] = jnp.zeros((dst.shape[1] - mh, 128), F32)

    stage(x1s, x2s, b1a_ref, b1b_ref, t1_ref)
    stage(x2s, x3s, b2a_ref, b2b_ref, t2_ref)
    stage(x3s, x4s, b3a_ref, b3b_ref, t3_ref)
    stage(x4s, ps, b4a_ref, b4b_ref, t4_ref)

    feats = jnp.concatenate(
        [ps[wp, pl.Slice(hp, nb, 4), :]
         for hp in range(2) for wp in range(2)], axis=1)
    hid = jnp.maximum(
        jnp.dot(feats, wf1_ref[...], preferred_element_type=F32) + bf1_ref[...], 0.0)
    logits = jnp.dot(hid, wf2_ref[...], preferred_element_type=F32) + bf2_ref[...]
    m = jnp.max(logits, axis=1, keepdims=True)
    lse = m + jnp.log(jnp.sum(jnp.exp(logits - m), axis=1, keepdims=True))
    out_ref[...] = logits - lse


def kernel(x_nchw, w1k, b1k, w2k, b2k, w3k, b3k, w4k, b4k, wf1k, bf1k, wf2k, bf2k):
    n, c_in, h, w = x_nchw.shape
    nb = 8
    assert n % nb == 0 and (h, w, c_in) == (64, 64, 3)
    nc = wf2k.shape[-1]

    x2d = jnp.transpose(x_nchw, (0, 2, 3, 1)).astype(F32).reshape(n * h, w * c_in)
    x3d = jnp.pad(x2d, ((0, 0), (0, 64))).reshape(n * h, 2, 128).transpose(1, 0, 2)

    b1a = _band(w1k, 3, 16, 64, 32, 0, k_piece_pad=256)
    b1b = _band(w1k, 3, 16, 64, 32, 1, k_piece_pad=256)
    b2a = _band(w2k, 16, 32, 32, 16, 0)
    b2b = _band(w2k, 16, 32, 32, 16, 1)
    b3a = _band(w3k, 32, 64, 16, 8, 0)
    b3b = _band(w3k, 32, 64, 16, 8, 1)
    b4a = _band(w4k, 64, 128, 8, 2, 0)
    b4b = _band(w4k, 64, 128, 8, 2, 1)
    t1 = jnp.tile(b1k, (1, 32))
    t2 = jnp.tile(b2k, (1, 16))
    t3 = jnp.tile(b3k, (1, 8))
    t4 = jnp.tile(b4k, (1, 2))
    wf1 = wf1k.reshape(4 * 128, 128)

    others = (b1a, b1b, t1, b2a, b2b, t2, b3a, b3b, t3, b4a, b4b, t4,
              wf1, bf1k, wf2k, bf2k)

    def full(a):
        return pl.BlockSpec(a.shape, lambda i, nd=a.ndim: (0,) * nd)

    out = pl.pallas_call(
        _net_kernel,
        out_shape=jax.ShapeDtypeStruct((n // nb, nb, nc), F32),
        grid_spec=pltpu.PrefetchScalarGridSpec(
            num_scalar_prefetch=0,
            grid=(n // nb,),
            in_specs=[pl.BlockSpec((2, nb * h, 128), lambda i: (0, i, 0))]
                     + [full(a) for a in others],
            out_specs=pl.BlockSpec((pl.Squeezed(), nb, nc), lambda i: (i, 0, 0)),
            scratch_shapes=[
                pltpu.VMEM((2, nb * 64 + 8, 128), F32),
                pltpu.VMEM((4, nb * 32 + 8, 128), F32),
                pltpu.VMEM((4, nb * 16 + 8, 128), F32),
                pltpu.VMEM((4, nb * 8 + 8, 128), F32),
                pltpu.VMEM((2, nb * 4, 128), F32),
            ]),
        compiler_params=pltpu.CompilerParams(
            dimension_semantics=("parallel",),
            vmem_limit_bytes=60 << 20),
    )(x3d, *others)
    return out.reshape(n, nc)


# speedup vs baseline: 40.5682x; 1.2912x over previous
import numpy as np
import jax
import jax.numpy as jnp
from jax import lax
from jax.experimental import pallas as pl
from jax.experimental.pallas import tpu as pltpu

F32 = jnp.float32


def _band(wk, cin, cout, w_in_pad, w_out_pad, pw, k_piece_pad=None):
    k4 = wk.reshape(3, 3, cin, cout)
    sel = np.zeros((3, w_in_pad, w_out_pad), np.float32)
    for dw in range(3):
        for wp in range(w_out_pad):
            w = 2 * wp + pw + dw
            if w < w_in_pad:
                sel[dw, w, wp] = 1.0
    sel = jnp.asarray(sel)
    b = jnp.einsum('awp,haio->hwipo', sel, k4)
    b = b.reshape(3, w_in_pad * cin, w_out_pad * cout)
    if k_piece_pad is not None:
        b = jnp.pad(b, ((0, 0), (0, k_piece_pad - w_in_pad * cin), (0, 0)))
    return b.reshape(-1, w_out_pad * cout).astype(jnp.bfloat16)


def _net_kernel(x_ref,
                b1a_ref, b1b_ref, t1_ref,
                b2a_ref, b2b_ref, t2_ref,
                b3a_ref, b3b_ref, t3_ref,
                b4a_ref, b4b_ref, t4_ref,
                wf1_ref, bf1_ref, wf2_ref, bf2_ref,
                out_ref,
                x1s, x2s, x3s, x4s, ps):
    nb8 = x1s.shape[1] - 8
    nb = nb8 // 64

    x1s[:, 0:nb8, :] = x_ref[...]
    x1s[:, nb8:nb8 + 8, :] = jnp.zeros((2, 8, 128), F32)

    def stage(src, dst, ba_ref, bb_ref, bt_ref):
        ng = src.shape[0]
        mh = (src.shape[1] - 8) // 2

        def lhs_for(ph):
            return jnp.concatenate(
                [src[c, pl.Slice(ph + dh, mh, 2), :]
                 for dh in range(3) for c in range(ng)], axis=1)

        lhs = jnp.concatenate([lhs_for(0), lhs_for(1)], axis=0).astype(jnp.bfloat16)
        bt = bt_ref[...]
        acta = jnp.maximum(jnp.dot(lhs, ba_ref[...], preferred_element_type=F32) + bt, 0.0)
        actb = jnp.maximum(jnp.dot(lhs, bb_ref[...], preferred_element_type=F32) + bt, 0.0)
        act = jnp.maximum(acta, actb)
        pooled = jnp.maximum(act[0:mh, :], act[mh:2 * mh, :])
        for c in range(dst.shape[0]):
            dst[c, 0:mh, :] = pooled[:, c * 128:(c + 1) * 128]
            if dst.shape[1] > mh:
                dst[c, mh:, :] = jnp.zeros((dst.shape[1] - mh, 128), F32)

    stage(x1s, x2s, b1a_ref, b1b_ref, t1_ref)
    stage(x2s, x3s, b2a_ref, b2b_ref, t2_ref)
    stage(x3s, x4s, b3a_ref, b3b_ref, t3_ref)
    stage(x4s, ps, b4a_ref, b4b_ref, t4_ref)

    feats = jnp.concatenate(
        [ps[wp, pl.Slice(hp, nb, 4), :]
         for hp in range(2) for wp in range(2)], axis=1)
    hid = jnp.maximum(
        jnp.dot(feats, wf1_ref[...], preferred_element_type=F32) + bf1_ref[...], 0.0)
    logits = jnp.dot(hid, wf2_ref[...], preferred_element_type=F32) + bf2_ref[...]
    m = jnp.max(logits, axis=1, keepdims=True)
    lse = m + jnp.log(jnp.sum(jnp.exp(logits - m), axis=1, keepdims=True))
    out_ref[...] = logits - lse


def kernel(x_nchw, w1k, b1k, w2k, b2k, w3k, b3k, w4k, b4k, wf1k, bf1k, wf2k, bf2k):
    n, c_in, h, w = x_nchw.shape
    nb = 16
    assert n % nb == 0 and (h, w, c_in) == (64, 64, 3)
    nc = wf2k.shape[-1]

    x2d = jnp.transpose(x_nchw, (0, 2, 3, 1)).astype(F32).reshape(n * h, w * c_in)
    x3d = jnp.pad(x2d, ((0, 0), (0, 64))).reshape(n * h, 2, 128).transpose(1, 0, 2)

    b1a = _band(w1k, 3, 16, 64, 32, 0, k_piece_pad=256)
    b1b = _band(w1k, 3, 16, 64, 32, 1, k_piece_pad=256)
    b2a = _band(w2k, 16, 32, 32, 16, 0)
    b2b = _band(w2k, 16, 32, 32, 16, 1)
    b3a = _band(w3k, 32, 64, 16, 8, 0)
    b3b = _band(w3k, 32, 64, 16, 8, 1)
    b4a = _band(w4k, 64, 128, 8, 2, 0)
    b4b = _band(w4k, 64, 128, 8, 2, 1)
    t1 = jnp.tile(b1k, (1, 32))
    t2 = jnp.tile(b2k, (1, 16))
    t3 = jnp.tile(b3k, (1, 8))
    t4 = jnp.tile(b4k, (1, 2))
    wf1 = wf1k.reshape(4 * 128, 128)

    others = (b1a, b1b, t1, b2a, b2b, t2, b3a, b3b, t3, b4a, b4b, t4,
              wf1, bf1k, wf2k, bf2k)

    def full(a):
        return pl.BlockSpec(a.shape, lambda i, nd=a.ndim: (0,) * nd)

    out = pl.pallas_call(
        _net_kernel,
        out_shape=jax.ShapeDtypeStruct((n // nb, nb, nc), F32),
        grid_spec=pltpu.PrefetchScalarGridSpec(
            num_scalar_prefetch=0,
            grid=(n // nb,),
            in_specs=[pl.BlockSpec((2, nb * h, 128), lambda i: (0, i, 0))]
                     + [full(a) for a in others],
            out_specs=pl.BlockSpec((pl.Squeezed(), nb, nc), lambda i: (i, 0, 0)),
            scratch_shapes=[
                pltpu.VMEM((2, nb * 64 + 8, 128), F32),
                pltpu.VMEM((4, nb * 32 + 8, 128), F32),
                pltpu.VMEM((4, nb * 16 + 8, 128), F32),
                pltpu.VMEM((4, nb * 8 + 8, 128), F32),
                pltpu.VMEM((2, nb * 4, 128), F32),
            ]),
        compiler_params=pltpu.CompilerParams(
            dimension_semantics=("parallel",),
            vmem_limit_bytes=60 << 20),
    )(x3d, *others)
    return out.reshape(n, nc)
